# baseline (speedup 1.0000x reference)
import functools
import math

import jax
import jax.numpy as jnp
from jax import lax
from jax.experimental import pallas as pl
from jax.experimental.pallas import tpu as pltpu
from jax.experimental.pallas import tpu_sc as plsc

D_MODEL = 64
TOK = 64
PAIRS = TOK // 2
NBUF = 4
SCALE = math.sqrt(D_MODEL)


@functools.partial(jax.jit, static_argnames=("n_tok",))
def _embed(idx, table2, n_tok):
    info = plsc.get_sparse_core_info()
    nw = info.num_cores * info.num_subcores
    tok_per_w = n_tok // nw
    n_chunks = tok_per_w // TOK
    n_groups = n_chunks // NBUF
    assert n_groups % 2 == 0
    mesh = plsc.VectorSubcoreMesh(core_axis_name="c", subcore_axis_name="s")

    @functools.partial(
        pl.kernel,
        mesh=mesh,
        compiler_params=pltpu.CompilerParams(use_tc_tiling_on_sc=True),
        out_type=jax.ShapeDtypeStruct((n_tok // 2, 2 * D_MODEL), jnp.float32),
        scratch_types=[
            pltpu.VMEM((n_chunks, TOK), jnp.int32),
            pltpu.VMEM((2, NBUF, TOK), jnp.int32),
        ]
        + [
            pltpu.VMEM((TOK, 2 * D_MODEL), jnp.float32)
            for _ in range(2 * NBUF)
        ]
        + [pltpu.SemaphoreType.DMA for _ in range(4)],
    )
    def k(idx_hbm, table_hbm, out_hbm, idx_v, iv, *rest):
        bufs_a = rest[0:NBUF]
        bufs_b = rest[NBUF : 2 * NBUF]
        gsem_a, ssem_a, gsem_b, ssem_b = rest[2 * NBUF : 2 * NBUF + 4]

        cid = lax.axis_index("c")
        sid = lax.axis_index("s")
        wid = sid * info.num_cores + cid
        pair0 = wid * (tok_per_w // 2)
        pltpu.sync_copy(idx_hbm.at[wid], idx_v)

        def out_slice(c):
            return out_hbm.at[pl.ds(pair0 + c * PAIRS, PAIRS), :]

        def gather(c, s, b, buf, sem):
            for q in range(TOK // 16):
                sl = pl.ds(q * 16, 16)
                iv[s, b, sl] = lax.shift_right_logical(idx_v[c, sl], 1)
            pltpu.async_copy(table_hbm.at[iv.at[s, b]], buf, sem)

        def gather_wait(s, b, buf, sem):
            pltpu.make_async_copy(table_hbm.at[iv.at[s, b]], buf, sem).wait()

        def scatter(c, buf, sem):
            pltpu.async_copy(buf.at[pl.ds(0, PAIRS)], out_slice(c), sem)

        def scatter_wait(c, buf, sem):
            pltpu.make_async_copy(
                buf.at[pl.ds(0, PAIRS)], out_slice(c), sem
            ).wait()

        def assemble(c, buf):
            def body(q, carry):
                offs = (idx_v[c, pl.ds(q * 16, 16)] & 1) * D_MODEL
                for u in range(16):
                    t = q * 16 + u
                    kk = q * 8 + u // 2
                    off = offs[u]
                    for v in range(D_MODEL // 16):
                        src = buf[t, pl.ds(off + 16 * v, 16)]
                        dst = pl.ds((u % 2) * D_MODEL + 16 * v, 16)
                        buf[kk, dst] = src * SCALE
                return carry

            lax.fori_loop(0, TOK // 16, body, 0, unroll=False)

        for b in range(NBUF):
            gather(b, 0, b, bufs_a[b], gsem_a)

        def pair_body(p, carry):
            ga = 2 * p
            gb = 2 * p + 1

            for b in range(NBUF):
                gather(gb * NBUF + b, 1, b, bufs_b[b], gsem_b)

            for b in range(NBUF):
                c = ga * NBUF + b
                gather_wait(0, b, bufs_a[b], gsem_a)
                assemble(c, bufs_a[b])
                scatter(c, bufs_a[b], ssem_a)

            for b in range(NBUF):
                scatter_wait(ga * NBUF + b, bufs_a[b], ssem_a)

            @pl.when(ga + 2 < n_groups)
            def _():
                for b in range(NBUF):
                    gather((ga + 2) * NBUF + b, 0, b, bufs_a[b], gsem_a)

            for b in range(NBUF):
                c = gb * NBUF + b
                gather_wait(1, b, bufs_b[b], gsem_b)
                assemble(c, bufs_b[b])
                scatter(c, bufs_b[b], ssem_b)

            for b in range(NBUF):
                scatter_wait(gb * NBUF + b, bufs_b[b], ssem_b)

            return carry

        lax.fori_loop(0, n_groups // 2, pair_body, 0)

    return k(idx, table2)


def kernel(x, table):
    n_seq, seq_len = x.shape
    n_tok = n_seq * seq_len
    info = plsc.get_sparse_core_info()
    nw = info.num_cores * info.num_subcores
    idx = x.astype(jnp.int32).reshape(nw, n_tok // nw // TOK, TOK)
    table2 = table.reshape(table.shape[0] // 2, 2 * D_MODEL)
    out2 = _embed(idx, table2, n_tok)
    return out2.reshape(n_seq, seq_len, D_MODEL)


# speedup vs baseline: 1.0774x; 1.0774x over previous
import functools
import math

import jax
import jax.numpy as jnp
from jax import lax
from jax.experimental import pallas as pl
from jax.experimental.pallas import tpu as pltpu
from jax.experimental.pallas import tpu_sc as plsc

D_MODEL = 64
TOK = 128
PAIRS = TOK // 2
NBUF = 2
SCALE = math.sqrt(D_MODEL)


@functools.partial(jax.jit, static_argnames=("n_tok",))
def _embed(idx, table2, n_tok):
    info = plsc.get_sparse_core_info()
    nw = info.num_cores * info.num_subcores
    tok_per_w = n_tok // nw
    n_chunks = tok_per_w // TOK
    n_groups = n_chunks // NBUF
    assert n_groups % 2 == 0
    mesh = plsc.VectorSubcoreMesh(core_axis_name="c", subcore_axis_name="s")

    @functools.partial(
        pl.kernel,
        mesh=mesh,
        compiler_params=pltpu.CompilerParams(use_tc_tiling_on_sc=True),
        out_type=jax.ShapeDtypeStruct((n_tok // 2, 2 * D_MODEL), jnp.float32),
        scratch_types=[
            pltpu.VMEM((n_chunks, TOK), jnp.int32),
            pltpu.VMEM((2, NBUF, TOK), jnp.int32),
        ]
        + [
            pltpu.VMEM((TOK, 2 * D_MODEL), jnp.float32)
            for _ in range(2 * NBUF)
        ]
        + [pltpu.SemaphoreType.DMA for _ in range(4)],
    )
    def k(idx_hbm, table_hbm, out_hbm, idx_v, iv, *rest):
        bufs_a = rest[0:NBUF]
        bufs_b = rest[NBUF : 2 * NBUF]
        gsem_a, ssem_a, gsem_b, ssem_b = rest[2 * NBUF : 2 * NBUF + 4]

        cid = lax.axis_index("c")
        sid = lax.axis_index("s")
        wid = sid * info.num_cores + cid
        pair0 = wid * (tok_per_w // 2)
        pltpu.sync_copy(idx_hbm.at[wid], idx_v)

        def out_slice(c):
            return out_hbm.at[pl.ds(pair0 + c * PAIRS, PAIRS), :]

        def gather(c, s, b, buf, sem):
            for q in range(TOK // 16):
                sl = pl.ds(q * 16, 16)
                iv[s, b, sl] = lax.shift_right_logical(idx_v[c, sl], 1)
            pltpu.async_copy(table_hbm.at[iv.at[s, b]], buf, sem)

        def gather_wait(s, b, buf, sem):
            pltpu.make_async_copy(table_hbm.at[iv.at[s, b]], buf, sem).wait()

        def scatter(c, buf, sem):
            pltpu.async_copy(buf.at[pl.ds(0, PAIRS)], out_slice(c), sem)

        def scatter_wait(c, buf, sem):
            pltpu.make_async_copy(
                buf.at[pl.ds(0, PAIRS)], out_slice(c), sem
            ).wait()

        def assemble(c, buf):
            def body(q, carry):
                offs = (idx_v[c, pl.ds(q * 16, 16)] & 1) * D_MODEL
                for u in range(16):
                    t = q * 16 + u
                    kk = q * 8 + u // 2
                    off = offs[u]
                    for v in range(D_MODEL // 16):
                        src = buf[t, pl.ds(off + 16 * v, 16)]
                        dst = pl.ds((u % 2) * D_MODEL + 16 * v, 16)
                        buf[kk, dst] = src * SCALE
                return carry

            lax.fori_loop(0, TOK // 16, body, 0, unroll=False)

        for b in range(NBUF):
            gather(b, 0, b, bufs_a[b], gsem_a)

        def pair_body(p, carry):
            ga = 2 * p
            gb = 2 * p + 1

            for b in range(NBUF):
                gather(gb * NBUF + b, 1, b, bufs_b[b], gsem_b)

            for b in range(NBUF):
                c = ga * NBUF + b
                gather_wait(0, b, bufs_a[b], gsem_a)
                assemble(c, bufs_a[b])
                scatter(c, bufs_a[b], ssem_a)

            for b in range(NBUF):
                scatter_wait(ga * NBUF + b, bufs_a[b], ssem_a)

            @pl.when(ga + 2 < n_groups)
            def _():
                for b in range(NBUF):
                    gather((ga + 2) * NBUF + b, 0, b, bufs_a[b], gsem_a)

            for b in range(NBUF):
                c = gb * NBUF + b
                gather_wait(1, b, bufs_b[b], gsem_b)
                assemble(c, bufs_b[b])
                scatter(c, bufs_b[b], ssem_b)

            for b in range(NBUF):
                scatter_wait(gb * NBUF + b, bufs_b[b], ssem_b)

            return carry

        lax.fori_loop(0, n_groups // 2, pair_body, 0)

    return k(idx, table2)


def kernel(x, table):
    n_seq, seq_len = x.shape
    n_tok = n_seq * seq_len
    info = plsc.get_sparse_core_info()
    nw = info.num_cores * info.num_subcores
    idx = x.astype(jnp.int32).reshape(nw, n_tok // nw // TOK, TOK)
    table2 = table.reshape(table.shape[0] // 2, 2 * D_MODEL)
    out2 = _embed(idx, table2, n_tok)
    return out2.reshape(n_seq, seq_len, D_MODEL)
